# Initial kernel scaffold; baseline (speedup 1.0000x reference)
import jax
import jax.numpy as jnp
from jax.experimental import pallas as pl


def kernel(x, W):
    raise NotImplementedError("write your pallas kernel here")


# speedup vs baseline: 3.6840x; 3.6840x over previous
import functools

import jax
import jax.numpy as jnp
import numpy as np
from jax import lax
from jax.experimental import pallas as pl
from jax.experimental.pallas import tpu as pltpu
from jax.experimental.pallas import tpu_sc as plsc

_EMBED = 128
_LANES = 16
_NUM_WORKERS = 32


def _make_pe(maxlen: int, d: int) -> np.ndarray:
    pe = np.zeros((maxlen, d), dtype=np.float32)
    position = np.arange(0, maxlen)[:, np.newaxis]
    div_term = np.exp(np.arange(0, d, 2) * -(np.log(10000.0) / d))
    pe[:, 0::2] = np.sin(position * div_term)
    pe[:, 1::2] = np.cos(position * div_term)
    return pe


@functools.partial(jax.jit, static_argnums=(2, 3))
def _embed_fixed(x_flat, w, b, l):
    d = w.shape[1]
    n = b * l
    rows_per_w = n // _NUM_WORKERS
    seq_per_w = b // _NUM_WORKERS
    chunks = [(0, 128), (128, l - 128)] if l > 128 else [(0, l)]
    pe = jnp.asarray(_make_pe(l, d))

    mesh = plsc.VectorSubcoreMesh(core_axis_name="c", subcore_axis_name="s")

    @functools.partial(
        pl.kernel,
        out_type=jax.ShapeDtypeStruct((n, d), jnp.float32),
        mesh=mesh,
        scratch_types=[
            pltpu.VMEM((rows_per_w,), jnp.int32),
            pltpu.VMEM((l, d), jnp.float32),
            pltpu.VMEM((chunks[0][1], d), jnp.float32),
            pltpu.VMEM((chunks[-1][1], d), jnp.float32),
            pltpu.SemaphoreType.DMA,
        ],
    )
    def run(x_hbm, pe_hbm, w_hbm, out_hbm, idx_v, pe_v, buf0, buf1, sem):
        wid = lax.axis_index("s") * 2 + lax.axis_index("c")
        base = wid * rows_per_w
        pltpu.sync_copy(x_hbm.at[pl.ds(base, rows_per_w)], idx_v)
        pltpu.sync_copy(pe_hbm, pe_v)
        bufs = [buf0, buf1]

        @pl.loop(0, seq_per_w)
        def _seq(s):
            row0 = s * l
            for ci, (off, sz) in enumerate(chunks):
                buf = bufs[ci]
                pltpu.async_copy(
                    w_hbm.at[idx_v.at[pl.ds(row0 + off, sz)]], buf, sem
                ).wait()

                @pl.loop(0, sz)
                def _row(r):
                    for j in range(d // _LANES):
                        c = pl.ds(j * _LANES, _LANES)
                        buf[r, c] += pe_v[off + r, c]

                pltpu.sync_copy(
                    buf, out_hbm.at[pl.ds(base + row0 + off, sz)]
                )

    return run(x_flat, pe, w)


def kernel(x, W):
    b, l = x.shape
    d = W.shape[1]
    out = _embed_fixed(x.reshape(b * l), W, b, l)
    return out.reshape(b, l, d)


# speedup vs baseline: 6.2557x; 1.6981x over previous
import functools

import jax
import jax.numpy as jnp
import numpy as np
from jax import lax
from jax.experimental import pallas as pl
from jax.experimental.pallas import tpu as pltpu
from jax.experimental.pallas import tpu_sc as plsc

_EMBED = 128
_LANES = 16
_NUM_WORKERS = 32


def _make_pe(maxlen: int, d: int) -> np.ndarray:
    pe = np.zeros((maxlen, d), dtype=np.float32)
    position = np.arange(0, maxlen)[:, np.newaxis]
    div_term = np.exp(np.arange(0, d, 2) * -(np.log(10000.0) / d))
    pe[:, 0::2] = np.sin(position * div_term)
    pe[:, 1::2] = np.cos(position * div_term)
    return pe


@functools.partial(jax.jit, static_argnums=(2, 3))
def _embed_fixed(x_flat, w, b, l):
    d = w.shape[1]
    n = b * l
    rows_per_w = n // _NUM_WORKERS
    seq_per_w = b // _NUM_WORKERS
    chunks = [(0, 128), (128, l - 128)] if l > 128 else [(0, l)]
    pe = jnp.asarray(_make_pe(l, d))

    mesh = plsc.VectorSubcoreMesh(core_axis_name="c", subcore_axis_name="s")

    @functools.partial(
        pl.kernel,
        out_type=jax.ShapeDtypeStruct((n, d), jnp.float32),
        mesh=mesh,
        scratch_types=[
            pltpu.VMEM((rows_per_w,), jnp.int32),
            pltpu.VMEM((l, d), jnp.float32),
            pltpu.VMEM((chunks[0][1], d), jnp.float32),
            pltpu.VMEM((chunks[-1][1], d), jnp.float32),
            pltpu.SemaphoreType.DMA,
            pltpu.SemaphoreType.DMA,
        ],
    )
    def run(x_hbm, pe_hbm, w_hbm, out_hbm, idx_v, pe_v, buf0, buf1, sem0, sem1):
        wid = lax.axis_index("s") * 2 + lax.axis_index("c")
        base = wid * rows_per_w
        pltpu.sync_copy(x_hbm.at[pl.ds(base, rows_per_w)], idx_v)
        pltpu.sync_copy(pe_hbm, pe_v)
        bufs = [buf0, buf1]
        sems = [sem0, sem1]

        def fill_and_gather(s, ci):
            off, sz = chunks[ci]
            buf = bufs[ci]

            @pl.loop(0, sz)
            def _row(r):
                for j in range(d // _LANES):
                    c = pl.ds(j * _LANES, _LANES)
                    buf[r, c] = pe_v[off + r, c]

            pltpu.async_copy(
                w_hbm.at[idx_v.at[pl.ds(s * l + off, sz)]],
                buf,
                sems[ci],
                add=True,
            )

        def wait_and_store(s, ci):
            off, sz = chunks[ci]
            pltpu.make_async_copy(
                w_hbm.at[idx_v.at[pl.ds(s * l + off, sz)]], bufs[ci], sems[ci]
            ).wait()
            pltpu.sync_copy(bufs[ci], out_hbm.at[pl.ds(base + s * l + off, sz)])

        fill_and_gather(0, 0)

        @pl.loop(0, seq_per_w)
        def _seq(s):
            fill_and_gather(s, 1)
            wait_and_store(s, 0)

            @pl.when(s + 1 < seq_per_w)
            def _():
                fill_and_gather(s + 1, 0)

            wait_and_store(s, 1)

    return run(x_flat, pe, w)


def kernel(x, W):
    b, l = x.shape
    d = W.shape[1]
    out = _embed_fixed(x.reshape(b * l), W, b, l)
    return out.reshape(b, l, d)
